# Initial kernel scaffold; baseline (speedup 1.0000x reference)
import jax
import jax.numpy as jnp
from jax.experimental import pallas as pl


def kernel(x, edge_index, edge_attr, lin_in_W, lin_in_b, conv_W, att_src, att_dst, edge_W, att_edge, conv_b, pred_W, pred_b):
    raise NotImplementedError("write your pallas kernel here")


# speedup vs baseline: 44.4022x; 44.4022x over previous
import functools

import jax
import jax.numpy as jnp
from jax import lax
from jax.experimental import pallas as pl
from jax.experimental.pallas import tpu as pltpu
from jax.experimental.pallas import tpu_sc as plsc

N = 10000
E = 320000
D_IN = 128
EMB = 64
H = 8
C = 8
L = 4
HP = 16

BN = 1000
BE = 4000

NC = 2
NS = 16
EPT = E // (NC * NS)
CK = 80
NCH = EPT // CK
RZ = 80
NRCH = N // RZ


def _pre_kernel(x, lin_W, lin_b, W0, Ms, Md):
  def body(x_r, W_r, b_r, W0_r, Ms_r, Md_r, nt_o):
    h = jnp.dot(x_r[...], W_r[...], preferred_element_type=jnp.float32)
    h = h + b_r[...]
    hl = jnp.dot(h, W0_r[...], preferred_element_type=jnp.float32)
    a_s = jnp.dot(hl, Ms_r[...], preferred_element_type=jnp.float32)
    a_d = jnp.dot(hl, Md_r[...], preferred_element_type=jnp.float32)
    nt_o[...] = jnp.concatenate([hl, a_s, a_d], axis=1)

  return pl.pallas_call(
      body,
      grid=(N // BN,),
      in_specs=[
          pl.BlockSpec((BN, D_IN), lambda i: (i, 0)),
          pl.BlockSpec((D_IN, EMB), lambda i: (0, 0)),
          pl.BlockSpec((1, EMB), lambda i: (0, 0)),
          pl.BlockSpec((EMB, EMB), lambda i: (0, 0)),
          pl.BlockSpec((EMB, HP), lambda i: (0, 0)),
          pl.BlockSpec((EMB, HP), lambda i: (0, 0)),
      ],
      out_specs=pl.BlockSpec((BN, 96), lambda i: (i, 0)),
      out_shape=jax.ShapeDtypeStruct((N, 96), jnp.float32),
  )(x, lin_W, lin_b, W0, Ms, Md)


def _aedge_kernel(edge_attr, Wae):
  def body(ea_r, W_r, o_r):
    o_r[...] = jnp.dot(ea_r[...], W_r[...], preferred_element_type=jnp.float32)

  return pl.pallas_call(
      body,
      grid=(E // BE,),
      in_specs=[
          pl.BlockSpec((BE, 16), lambda i: (i, 0)),
          pl.BlockSpec((16, L * HP), lambda i: (0, 0)),
      ],
      out_specs=pl.BlockSpec((BE, L * HP), lambda i: (i, 0)),
      out_shape=jax.ShapeDtypeStruct((E, L * HP), jnp.float32),
  )(edge_attr, Wae)


def _mid_kernel(p0, p1, ExpM, cb, Wn, Ms, Md):
  def body(p0_r, p1_r, E_r, cb_r, Wn_r, Ms_r, Md_r, nt_o):
    a0_r = p0_r[:, 0:64]
    a1_r = p1_r[:, 0:64]
    den = p0_r[:, 64:80] + p1_r[:, 64:80]
    rden = 1.0 / (den + 1e-16)
    rex = jnp.dot(rden, E_r[...], preferred_element_type=jnp.float32)
    h = (a0_r + a1_r) * rex + cb_r[...]
    hl = jnp.dot(h, Wn_r[...], preferred_element_type=jnp.float32)
    a_s = jnp.dot(hl, Ms_r[...], preferred_element_type=jnp.float32)
    a_d = jnp.dot(hl, Md_r[...], preferred_element_type=jnp.float32)
    nt_o[...] = jnp.concatenate([hl, a_s, a_d], axis=1)

  return pl.pallas_call(
      body,
      grid=(N // BN,),
      in_specs=[
          pl.BlockSpec((BN, 80), lambda i: (i, 0)),
          pl.BlockSpec((BN, 80), lambda i: (i, 0)),
          pl.BlockSpec((HP, EMB), lambda i: (0, 0)),
          pl.BlockSpec((1, EMB), lambda i: (0, 0)),
          pl.BlockSpec((EMB, EMB), lambda i: (0, 0)),
          pl.BlockSpec((EMB, HP), lambda i: (0, 0)),
          pl.BlockSpec((EMB, HP), lambda i: (0, 0)),
      ],
      out_specs=pl.BlockSpec((BN, 96), lambda i: (i, 0)),
      out_shape=jax.ShapeDtypeStruct((N, 96), jnp.float32),
  )(p0, p1, ExpM, cb, Wn, Ms, Md)


def _final_kernel(p0, p1, ExpM, cb, pW, pb):
  def body(p0_r, p1_r, E_r, cb_r, pW_r, pb_r, o_r):
    den = p0_r[:, 64:80] + p1_r[:, 64:80]
    rden = 1.0 / (den + 1e-16)
    rex = jnp.dot(rden, E_r[...], preferred_element_type=jnp.float32)
    h = (p0_r[:, 0:64] + p1_r[:, 0:64]) * rex + cb_r[...]
    o_r[...] = jnp.dot(h, pW_r[...], preferred_element_type=jnp.float32) + pb_r[...]

  return pl.pallas_call(
      body,
      grid=(N // BN,),
      in_specs=[
          pl.BlockSpec((BN, 80), lambda i: (i, 0)),
          pl.BlockSpec((BN, 80), lambda i: (i, 0)),
          pl.BlockSpec((HP, EMB), lambda i: (0, 0)),
          pl.BlockSpec((1, EMB), lambda i: (0, 0)),
          pl.BlockSpec((EMB, 8), lambda i: (0, 0)),
          pl.BlockSpec((1, 8), lambda i: (0, 0)),
      ],
      out_specs=pl.BlockSpec((BN, 8), lambda i: (i, 0)),
      out_shape=jax.ShapeDtypeStruct((N, 8), jnp.float32),
  )(p0, p1, ExpM, cb, pW, pb)


def _edge_tc(srar, drar, ae, ExpM):
  def body(s_r, d_r, ae_r, E_r, o_r):
    v = s_r[:, 64:80] + d_r[:, 80:96] + ae_r[...]
    v = jnp.where(v > 0.0, v, 0.2 * v)
    ev = jnp.exp(v)
    rep = jnp.dot(ev, E_r[...], preferred_element_type=jnp.float32)
    o_r[...] = jnp.concatenate([s_r[:, 0:64] * rep, ev], axis=1)

  return pl.pallas_call(
      body,
      grid=(E // BE,),
      in_specs=[
          pl.BlockSpec((BE, 96), lambda i: (i, 0)),
          pl.BlockSpec((BE, 96), lambda i: (i, 0)),
          pl.BlockSpec((BE, HP), lambda i: (i, 0)),
          pl.BlockSpec((HP, EMB), lambda i: (0, 0)),
      ],
      out_specs=pl.BlockSpec((BE, 80), lambda i: (i, 0)),
      out_shape=jax.ShapeDtypeStruct((E, 80), jnp.float32),
  )(srar, drar, ae, ExpM)


NTC = N // RZ


def _sc_gather_body(nt_h, src_h, dst_h, sr_o, dr_o,
                    srcidx, dstidx, srows, drows, nt_sh, sem):
  c = lax.axis_index("c")
  s = lax.axis_index("s")
  wid = c * NS + s

  for k in range((NTC + NS - 1) // NS):
    cid = s + NS * k
    if (k + 1) * NS <= NTC:
      pltpu.sync_copy(nt_h.at[pl.ds(cid * RZ, RZ)],
                      nt_sh.at[pl.ds(cid * RZ, RZ)])
    else:
      @pl.when(cid < NTC)
      def _():
        pltpu.sync_copy(nt_h.at[pl.ds(cid * RZ, RZ)],
                        nt_sh.at[pl.ds(cid * RZ, RZ)])
  plsc.subcore_barrier()

  ebase = wid * EPT

  def chunk(i, carry):
    b0 = ebase + i * CK
    pltpu.sync_copy(src_h.at[pl.ds(b0, CK)], srcidx)
    pltpu.sync_copy(dst_h.at[pl.ds(b0, CK)], dstidx)
    cp1 = pltpu.async_copy(nt_sh.at[srcidx], srows, sem)
    cp2 = pltpu.async_copy(nt_sh.at[dstidx], drows, sem)
    cp1.wait()
    cp2.wait()
    pltpu.sync_copy(srows, sr_o.at[pl.ds(b0, CK)])
    pltpu.sync_copy(drows, dr_o.at[pl.ds(b0, CK)])
    return carry
  lax.fori_loop(0, NCH, chunk, None)


@functools.cache
def _sc_gather_call():
  return pl.kernel(
      _sc_gather_body,
      out_type=[
          jax.ShapeDtypeStruct((E, 96), jnp.float32),
          jax.ShapeDtypeStruct((E, 96), jnp.float32),
      ],
      mesh=plsc.VectorSubcoreMesh(core_axis_name="c", subcore_axis_name="s",
                                  num_cores=NC, num_subcores=NS),
      scratch_types=[
          pltpu.VMEM((CK,), jnp.int32),
          pltpu.VMEM((CK,), jnp.int32),
          pltpu.VMEM((CK, 96), jnp.float32),
          pltpu.VMEM((CK, 96), jnp.float32),
          pltpu.VMEM_SHARED((N, 96), jnp.float32),
          pltpu.SemaphoreType.DMA,
      ],
  )


def _sc_scatter_body(msg_h, dst_h, zr_h, acc_o,
                     dstidx, msgb, acc_sh, sem):
  c = lax.axis_index("c")
  s = lax.axis_index("s")
  wid = c * NS + s

  for k in range((NTC + NS - 1) // NS):
    cid = s + NS * k
    if (k + 1) * NS <= NTC:
      pltpu.sync_copy(zr_h, acc_sh.at[pl.ds(cid * RZ, RZ)])
    else:
      @pl.when(cid < NTC)
      def _():
        pltpu.sync_copy(zr_h, acc_sh.at[pl.ds(cid * RZ, RZ)])
  plsc.subcore_barrier()

  ebase = wid * EPT

  def chunk(i, carry):
    b0 = ebase + i * CK
    pltpu.sync_copy(dst_h.at[pl.ds(b0, CK)], dstidx)
    pltpu.sync_copy(msg_h.at[pl.ds(b0, CK)], msgb)
    pltpu.sync_copy(msgb, acc_sh.at[dstidx], add=True)
    return carry
  lax.fori_loop(0, NCH, chunk, None)

  plsc.subcore_barrier()
  for k in range((NTC + NS - 1) // NS):
    cid = s + NS * k
    if (k + 1) * NS <= NTC:
      rr = cid * RZ
      pltpu.sync_copy(acc_sh.at[pl.ds(rr, RZ)], acc_o.at[c, pl.ds(rr, RZ)])
    else:
      @pl.when(cid < NTC)
      def _():
        rr = cid * RZ
        pltpu.sync_copy(acc_sh.at[pl.ds(rr, RZ)], acc_o.at[c, pl.ds(rr, RZ)])


@functools.cache
def _sc_scatter_call():
  return pl.kernel(
      _sc_scatter_body,
      out_type=jax.ShapeDtypeStruct((NC, N, 80), jnp.float32),
      mesh=plsc.VectorSubcoreMesh(core_axis_name="c", subcore_axis_name="s",
                                  num_cores=NC, num_subcores=NS),
      scratch_types=[
          pltpu.VMEM((CK,), jnp.int32),
          pltpu.VMEM((CK, 80), jnp.float32),
          pltpu.VMEM_SHARED((N, 80), jnp.float32),
          pltpu.SemaphoreType.DMA,
      ],
  )


def _sc_edge(nt, ae_l, src32, dst32, zr):
  sr, dr = _sc_gather_call()(nt, src32, dst32)
  msg = _edge_tc(sr, dr, ae_l, jnp.concatenate(
      [jnp.kron(jnp.eye(H, dtype=jnp.float32), jnp.ones((1, C), jnp.float32)),
       jnp.zeros((H, EMB), jnp.float32)], axis=0))
  return _sc_scatter_call()(msg, dst32, zr)


def kernel(x, edge_index, edge_attr, lin_in_W, lin_in_b, conv_W, att_src,
           att_dst, edge_W, att_edge, conv_b, pred_W, pred_b):
  f32 = jnp.float32
  src32 = edge_index[0].astype(jnp.int32)
  dst32 = edge_index[1].astype(jnp.int32)

  blk = jnp.kron(jnp.eye(H, dtype=f32), jnp.ones((C, 1), f32))
  pad8 = jnp.zeros((EMB, H), f32)
  Ms = [jnp.concatenate([att_src[l].reshape(EMB, 1) * blk, pad8], axis=1)
        for l in range(L)]
  Md = [jnp.concatenate([att_dst[l].reshape(EMB, 1) * blk, pad8], axis=1)
        for l in range(L)]
  Wae = (edge_W.reshape(L, 16, H, C) * att_edge[:, None]).sum(-1)
  Wae = jnp.concatenate([Wae, jnp.zeros((L, 16, H), f32)], axis=2)
  Wae = Wae.transpose(1, 0, 2).reshape(16, L * HP)
  ExpM = jnp.concatenate(
      [jnp.kron(jnp.eye(H, dtype=f32), jnp.ones((1, C), f32)),
       jnp.zeros((H, EMB), f32)], axis=0)

  lin_b2 = lin_in_b.reshape(1, EMB)
  cbs = [conv_b[l].reshape(1, EMB) for l in range(L)]
  pW = jnp.concatenate([pred_W, jnp.zeros((EMB, 7), f32)], axis=1)
  pb = jnp.concatenate([pred_b, jnp.zeros((7,), f32)]).reshape(1, 8)

  aedge_all = _aedge_kernel(edge_attr, Wae)
  zr = jnp.zeros((RZ, 80), f32)

  nt = _pre_kernel(x, lin_in_W, lin_b2, conv_W[0], Ms[0], Md[0])

  for l in range(L):
    ae_l = lax.slice_in_dim(aedge_all, l * HP, (l + 1) * HP, axis=1)
    acc = _sc_edge(nt, ae_l, src32, dst32, zr)
    if l + 1 < L:
      nt = _mid_kernel(acc[0], acc[1], ExpM,
                       cbs[l], conv_W[l + 1], Ms[l + 1], Md[l + 1])
    else:
      out = _final_kernel(acc[0], acc[1], ExpM, cbs[l], pW, pb)
  return out[:, 0]
